# baseline (speedup 1.0000x reference)
import functools

import jax
import jax.numpy as jnp
from jax import lax
from jax.experimental import pallas as pl
from jax.experimental.pallas import tpu as pltpu
from jax.experimental.pallas import tpu_sc as plsc

B = 128
LT = 64
LI = 197
D = 768
NC = 2
NS = 16
NW = NC * NS
RPW = B // NW
L = 16
BIG = 1 << 30
NQ = 8


def _row_argmax(simbuf, j):
    lanes = lax.broadcasted_iota(jnp.int32, (L,), 0)
    best_val = jnp.full((L,), -jnp.inf, dtype=jnp.float32)
    best_idx = jnp.zeros((L,), dtype=jnp.int32)
    for k in range(B // L):
        v = simbuf[j, pl.ds(k * L, L)]
        idxs = lanes + (k * L)
        better = v > best_val
        best_val = jnp.where(better, v, best_val)
        best_idx = jnp.where(better, idxs, best_idx)
    m = jnp.max(best_val)
    cand = jnp.where(best_val == m, best_idx, BIG)
    return jnp.min(cand)


def _sc_body(tattn_hbm, iattn_hbm, t2i_hbm, i2t_hbm,
             idx_out, out_tattn, out_iattn,
             sim_t2i_v, sim_i2t_v, myidx_v, gsem):
    c = lax.axis_index("c")
    s = lax.axis_index("s")
    wid = c * NS + s
    base = wid * RPW

    pltpu.sync_copy(t2i_hbm.at[pl.ds(base, RPW)], sim_t2i_v)
    pltpu.sync_copy(i2t_hbm.at[pl.ds(base, RPW)], sim_i2t_v)

    img_idx = [_row_argmax(sim_t2i_v, j) for j in range(RPW)]
    txt_idx = [_row_argmax(sim_i2t_v, j) for j in range(RPW)]

    lanes = lax.broadcasted_iota(jnp.int32, (L,), 0)
    vec = jnp.zeros((L,), dtype=jnp.int32)
    for j in range(RPW):
        vec = jnp.where(lanes == j, img_idx[j], vec)
        vec = jnp.where(lanes == RPW + j, txt_idx[j], vec)
    myidx_v[...] = vec
    copies = [pltpu.async_copy(myidx_v, idx_out.at[wid], gsem)]

    for j in range(RPW):
        r = base + j
        copies.append(pltpu.async_copy(iattn_hbm.at[r], out_iattn.at[r], gsem))
        copies.append(pltpu.async_copy(tattn_hbm.at[r], out_tattn.at[r], gsem))
        copies.append(
            pltpu.async_copy(iattn_hbm.at[img_idx[j]], out_iattn.at[B + r], gsem))
        copies.append(
            pltpu.async_copy(tattn_hbm.at[txt_idx[j]], out_tattn.at[B + r], gsem))
    for cp in copies:
        cp.wait()


RPB = 8
NCOPY = B // RPB


def _sc_text_body(text_hbm, t2i_hbm, i2t_hbm, out_text,
                  sim_t2i_v, sim_i2t_v, buf0, buf1,
                  isem0, isem1, osem0, osem1):
    c = lax.axis_index("c")
    s = lax.axis_index("s")
    wid = c * NS + s
    base = wid * RPW

    pltpu.sync_copy(i2t_hbm.at[pl.ds(base, RPW)], sim_i2t_v)
    txt_idx = [_row_argmax(sim_i2t_v, j) for j in range(RPW)]

    moves = []
    for j in range(RPW):
        r = base + j
        moves.append((text_hbm.at[r], out_text.at[r]))
        moves.append((text_hbm.at[txt_idx[j]], out_text.at[B + r]))

    bufs = (buf0, buf1)
    isems = (isem0, isem1)
    osems = (osem0, osem1)
    pending = [None, None]
    for i, (src, dst) in enumerate(moves):
        b = i % 2
        if pending[b] is not None:
            pending[b].wait()
        pltpu.async_copy(src, bufs[b], isems[b]).wait()
        pending[b] = pltpu.async_copy(bufs[b], dst, osems[b])
    for h in pending:
        h.wait()


def _tc_copy_body(img_in, img_out, sem_i):
    i = pl.program_id(0)
    handles = []
    for j in range(RPB // 2):
        ci = pltpu.make_async_copy(
            img_in.at[:, pl.ds(i * RPB + 2 * j, 2), :],
            img_out.at[:, pl.ds(2 * j, 2), :], sem_i.at[j])
        ci.start()
        handles.append(ci)
    for h in handles:
        h.wait()


def _tc_gather_body(iref, img_in, img_prev, img_out, sem_i):
    i = pl.program_id(0)
    handles = []
    for j in range(RPB):
        n = i * RPB + j
        w = n // RPW
        sl = n % RPW
        hi = pltpu.make_async_copy(
            img_in.at[:, iref[w, sl], :], img_out.at[:, j, :],
            sem_i.at[j])
        hi.start()
        handles.append(hi)
    for h in handles:
        h.wait()


def _out_map(i, iref):
    return (i, 0, 0)


@jax.jit
def kernel(text_embeddings, text_attns, image_embeddings, image_attns,
           sim_t2i, sim_i2t):
    mesh = plsc.VectorSubcoreMesh(
        core_axis_name="c", subcore_axis_name="s",
        num_cores=NC, num_subcores=NS)
    sc_out_type = (
        jax.ShapeDtypeStruct((NW, L), jnp.int32),
        jax.ShapeDtypeStruct((2 * B, LT), jnp.float32),
        jax.ShapeDtypeStruct((2 * B, LI), jnp.float32),
    )
    sc_scratch = [
        pltpu.VMEM((RPW, B), jnp.float32),
        pltpu.VMEM((RPW, B), jnp.float32),
        pltpu.VMEM((L,), jnp.int32),
        pltpu.SemaphoreType.DMA,
    ]
    sc_run = pl.kernel(
        _sc_body, out_type=sc_out_type, mesh=mesh, scratch_types=sc_scratch,
        compiler_params=pltpu.CompilerParams(needs_layout_passes=False),
        name="neg_collector_sc_idx_attn")
    idx_packed, text_attns_all, image_attns_all = sc_run(
        text_attns, image_attns, sim_t2i, sim_i2t)

    sc_txt_run = pl.kernel(
        _sc_text_body,
        out_type=jax.ShapeDtypeStruct((2 * B, LT, D), jnp.float32),
        mesh=mesh,
        scratch_types=[
            pltpu.VMEM((RPW, B), jnp.float32),
            pltpu.VMEM((RPW, B), jnp.float32),
            pltpu.VMEM((LT, D), jnp.float32),
            pltpu.VMEM((LT, D), jnp.float32),
            pltpu.SemaphoreType.DMA,
            pltpu.SemaphoreType.DMA,
            pltpu.SemaphoreType.DMA,
            pltpu.SemaphoreType.DMA,
        ],
        compiler_params=pltpu.CompilerParams(needs_layout_passes=False),
        name="neg_collector_sc_text")
    text_embed_all = sc_txt_run(text_embeddings, sim_t2i, sim_i2t)

    img_t = jnp.transpose(image_embeddings, (1, 0, 2))
    img_half_t = pl.pallas_call(
        _tc_copy_body,
        grid=(NCOPY,),
        in_specs=[pl.BlockSpec(memory_space=pl.ANY)],
        out_specs=[pl.BlockSpec((LI, RPB, D), lambda i: (0, i, 0))],
        out_shape=[jax.ShapeDtypeStruct((LI, 2 * B, D), jnp.float32)],
        scratch_shapes=[pltpu.SemaphoreType.DMA((RPB,))],
        name="neg_collector_tc_copy",
    )(img_t)[0]

    gather_spec = pltpu.PrefetchScalarGridSpec(
        num_scalar_prefetch=1,
        grid=(B // RPB,),
        in_specs=[
            pl.BlockSpec(memory_space=pl.ANY),
            pl.BlockSpec(memory_space=pl.ANY),
        ],
        out_specs=[
            pl.BlockSpec((LI, RPB, D), lambda i, iref: (0, NCOPY + i, 0)),
        ],
        scratch_shapes=[
            pltpu.SemaphoreType.DMA((RPB,)),
        ],
    )
    image_embed_all_t = pl.pallas_call(
        _tc_gather_body,
        grid_spec=gather_spec,
        out_shape=[jax.ShapeDtypeStruct((LI, 2 * B, D), jnp.float32)],
        input_output_aliases={2: 0},
        name="neg_collector_tc_gather",
    )(idx_packed, img_t, img_half_t)[0]
    image_embed_all = jnp.transpose(image_embed_all_t, (1, 0, 2))

    return (text_embed_all, text_attns_all, image_embed_all, image_attns_all)


# speedup vs baseline: 1.0396x; 1.0396x over previous
import functools

import jax
import jax.numpy as jnp
from jax import lax
from jax.experimental import pallas as pl
from jax.experimental.pallas import tpu as pltpu
from jax.experimental.pallas import tpu_sc as plsc

B = 128
LT = 64
LI = 197
D = 768
NC = 2
NS = 16
NW = NC * NS
RPW = B // NW
L = 16
BIG = 1 << 30
NQ = 8


def _row_argmax(simbuf, j):
    lanes = lax.broadcasted_iota(jnp.int32, (L,), 0)
    best_val = jnp.full((L,), -jnp.inf, dtype=jnp.float32)
    best_idx = jnp.zeros((L,), dtype=jnp.int32)
    for k in range(B // L):
        v = simbuf[j, pl.ds(k * L, L)]
        idxs = lanes + (k * L)
        better = v > best_val
        best_val = jnp.where(better, v, best_val)
        best_idx = jnp.where(better, idxs, best_idx)
    m = jnp.max(best_val)
    cand = jnp.where(best_val == m, best_idx, BIG)
    return jnp.min(cand)


def _sc_body(tattn_hbm, iattn_hbm, t2i_hbm, i2t_hbm,
             idx_out, out_tattn, out_iattn,
             sim_t2i_v, sim_i2t_v, myidx_v, gsem):
    c = lax.axis_index("c")
    s = lax.axis_index("s")
    wid = c * NS + s
    base = wid * RPW

    pltpu.sync_copy(t2i_hbm.at[pl.ds(base, RPW)], sim_t2i_v)
    pltpu.sync_copy(i2t_hbm.at[pl.ds(base, RPW)], sim_i2t_v)

    img_idx = [_row_argmax(sim_t2i_v, j) for j in range(RPW)]
    txt_idx = [_row_argmax(sim_i2t_v, j) for j in range(RPW)]

    lanes = lax.broadcasted_iota(jnp.int32, (L,), 0)
    vec = jnp.zeros((L,), dtype=jnp.int32)
    for j in range(RPW):
        vec = jnp.where(lanes == j, img_idx[j], vec)
        vec = jnp.where(lanes == RPW + j, txt_idx[j], vec)
    myidx_v[...] = vec
    copies = [pltpu.async_copy(myidx_v, idx_out.at[wid], gsem)]

    for j in range(RPW):
        r = base + j
        copies.append(pltpu.async_copy(iattn_hbm.at[r], out_iattn.at[r], gsem))
        copies.append(pltpu.async_copy(tattn_hbm.at[r], out_tattn.at[r], gsem))
        copies.append(
            pltpu.async_copy(iattn_hbm.at[img_idx[j]], out_iattn.at[B + r], gsem))
        copies.append(
            pltpu.async_copy(tattn_hbm.at[txt_idx[j]], out_tattn.at[B + r], gsem))
    for cp in copies:
        cp.wait()


RPB = 8
CPB = 16
NCOPY = B // RPB


def _sc_text_body(text_hbm, t2i_hbm, i2t_hbm, out_text,
                  sim_t2i_v, sim_i2t_v, buf0, buf1,
                  isem0, isem1, osem0, osem1):
    c = lax.axis_index("c")
    s = lax.axis_index("s")
    wid = c * NS + s
    base = wid * RPW

    pltpu.sync_copy(i2t_hbm.at[pl.ds(base, RPW)], sim_i2t_v)
    txt_idx = [_row_argmax(sim_i2t_v, j) for j in range(RPW)]

    moves = []
    for j in range(RPW):
        r = base + j
        moves.append((text_hbm.at[r], out_text.at[r]))
        moves.append((text_hbm.at[txt_idx[j]], out_text.at[B + r]))

    bufs = (buf0, buf1)
    isems = (isem0, isem1)
    osems = (osem0, osem1)
    pending = [None, None]
    for i, (src, dst) in enumerate(moves):
        b = i % 2
        if pending[b] is not None:
            pending[b].wait()
        pltpu.async_copy(src, bufs[b], isems[b]).wait()
        pending[b] = pltpu.async_copy(bufs[b], dst, osems[b])
    for h in pending:
        h.wait()


def _tc_copy_body(img_in, img_out, sem_i):
    i = pl.program_id(0)
    handles = []
    for j in range(4):
        ci = pltpu.make_async_copy(
            img_in.at[:, pl.ds(i * CPB + (CPB // 4) * j, CPB // 4), :],
            img_out.at[:, pl.ds((CPB // 4) * j, CPB // 4), :], sem_i.at[j])
        ci.start()
        handles.append(ci)
    for h in handles:
        h.wait()


def _tc_gather_body(iref, img_in, img_prev, img_out, sem_i):
    i = pl.program_id(0)
    handles = []
    for j in range(RPB):
        n = i * RPB + j
        w = n // RPW
        sl = n % RPW
        hi = pltpu.make_async_copy(
            img_in.at[:, iref[w, sl], :], img_out.at[:, j, :],
            sem_i.at[j])
        hi.start()
        handles.append(hi)
    for h in handles:
        h.wait()


def _out_map(i, iref):
    return (i, 0, 0)


@jax.jit
def kernel(text_embeddings, text_attns, image_embeddings, image_attns,
           sim_t2i, sim_i2t):
    mesh = plsc.VectorSubcoreMesh(
        core_axis_name="c", subcore_axis_name="s",
        num_cores=NC, num_subcores=NS)
    sc_out_type = (
        jax.ShapeDtypeStruct((NW, L), jnp.int32),
        jax.ShapeDtypeStruct((2 * B, LT), jnp.float32),
        jax.ShapeDtypeStruct((2 * B, LI), jnp.float32),
    )
    sc_scratch = [
        pltpu.VMEM((RPW, B), jnp.float32),
        pltpu.VMEM((RPW, B), jnp.float32),
        pltpu.VMEM((L,), jnp.int32),
        pltpu.SemaphoreType.DMA,
    ]
    sc_run = pl.kernel(
        _sc_body, out_type=sc_out_type, mesh=mesh, scratch_types=sc_scratch,
        compiler_params=pltpu.CompilerParams(needs_layout_passes=False),
        name="neg_collector_sc_idx_attn")
    idx_packed, text_attns_all, image_attns_all = sc_run(
        text_attns, image_attns, sim_t2i, sim_i2t)

    sc_txt_run = pl.kernel(
        _sc_text_body,
        out_type=jax.ShapeDtypeStruct((2 * B, LT, D), jnp.float32),
        mesh=mesh,
        scratch_types=[
            pltpu.VMEM((RPW, B), jnp.float32),
            pltpu.VMEM((RPW, B), jnp.float32),
            pltpu.VMEM((LT, D), jnp.float32),
            pltpu.VMEM((LT, D), jnp.float32),
            pltpu.SemaphoreType.DMA,
            pltpu.SemaphoreType.DMA,
            pltpu.SemaphoreType.DMA,
            pltpu.SemaphoreType.DMA,
        ],
        compiler_params=pltpu.CompilerParams(needs_layout_passes=False),
        name="neg_collector_sc_text")
    text_embed_all = sc_txt_run(text_embeddings, sim_t2i, sim_i2t)

    img_t = jnp.transpose(image_embeddings, (1, 0, 2))
    img_half_t = pl.pallas_call(
        _tc_copy_body,
        grid=(B // CPB,),
        in_specs=[pl.BlockSpec(memory_space=pl.ANY)],
        out_specs=[pl.BlockSpec((LI, CPB, D), lambda i: (0, i, 0))],
        out_shape=[jax.ShapeDtypeStruct((LI, 2 * B, D), jnp.float32)],
        scratch_shapes=[pltpu.SemaphoreType.DMA((4,))],
        name="neg_collector_tc_copy",
    )(img_t)[0]

    gather_spec = pltpu.PrefetchScalarGridSpec(
        num_scalar_prefetch=1,
        grid=(B // RPB,),
        in_specs=[
            pl.BlockSpec(memory_space=pl.ANY),
            pl.BlockSpec(memory_space=pl.ANY),
        ],
        out_specs=[
            pl.BlockSpec((LI, RPB, D), lambda i, iref: (0, NCOPY + i, 0)),
        ],
        scratch_shapes=[
            pltpu.SemaphoreType.DMA((RPB,)),
        ],
    )
    image_embed_all_t = pl.pallas_call(
        _tc_gather_body,
        grid_spec=gather_spec,
        out_shape=[jax.ShapeDtypeStruct((LI, 2 * B, D), jnp.float32)],
        input_output_aliases={2: 0},
        name="neg_collector_tc_gather",
    )(idx_packed, img_t, img_half_t)[0]
    image_embed_all = jnp.transpose(image_embed_all_t, (1, 0, 2))

    return (text_embed_all, text_attns_all, image_embed_all, image_attns_all)


# speedup vs baseline: 1.1152x; 1.0728x over previous
import functools

import jax
import jax.numpy as jnp
from jax import lax
from jax.experimental import pallas as pl
from jax.experimental.pallas import tpu as pltpu
from jax.experimental.pallas import tpu_sc as plsc

B = 128
LT = 64
LI = 197
D = 768
NC = 2
NS = 16
NW = NC * NS
RPW = B // NW
L = 16
BIG = 1 << 30
NQ = 8


def _row_argmax(simbuf, j):
    lanes = lax.broadcasted_iota(jnp.int32, (L,), 0)
    best_val = jnp.full((L,), -jnp.inf, dtype=jnp.float32)
    best_idx = jnp.zeros((L,), dtype=jnp.int32)
    for k in range(B // L):
        v = simbuf[j, pl.ds(k * L, L)]
        idxs = lanes + (k * L)
        better = v > best_val
        best_val = jnp.where(better, v, best_val)
        best_idx = jnp.where(better, idxs, best_idx)
    m = jnp.max(best_val)
    cand = jnp.where(best_val == m, best_idx, BIG)
    return jnp.min(cand)


def _sc_body(tattn_hbm, iattn_hbm, t2i_hbm, i2t_hbm,
             idx_out, out_tattn, out_iattn,
             sim_t2i_v, sim_i2t_v, myidx_v, gsem):
    c = lax.axis_index("c")
    s = lax.axis_index("s")
    wid = c * NS + s
    base = wid * RPW

    pltpu.sync_copy(t2i_hbm.at[pl.ds(base, RPW)], sim_t2i_v)
    pltpu.sync_copy(i2t_hbm.at[pl.ds(base, RPW)], sim_i2t_v)

    img_idx = [_row_argmax(sim_t2i_v, j) for j in range(RPW)]
    txt_idx = [_row_argmax(sim_i2t_v, j) for j in range(RPW)]

    lanes = lax.broadcasted_iota(jnp.int32, (L,), 0)
    vec = jnp.zeros((L,), dtype=jnp.int32)
    for j in range(RPW):
        vec = jnp.where(lanes == j, img_idx[j], vec)
        vec = jnp.where(lanes == RPW + j, txt_idx[j], vec)
    myidx_v[...] = vec
    copies = [pltpu.async_copy(myidx_v, idx_out.at[wid], gsem)]

    for j in range(RPW):
        r = base + j
        copies.append(pltpu.async_copy(iattn_hbm.at[r], out_iattn.at[r], gsem))
        copies.append(pltpu.async_copy(tattn_hbm.at[r], out_tattn.at[r], gsem))
        copies.append(
            pltpu.async_copy(iattn_hbm.at[img_idx[j]], out_iattn.at[B + r], gsem))
        copies.append(
            pltpu.async_copy(tattn_hbm.at[txt_idx[j]], out_tattn.at[B + r], gsem))
    for cp in copies:
        cp.wait()


RPB = 16
CPB = 32
NCOPY = B // RPB


def _sc_text_body(text_hbm, t2i_hbm, i2t_hbm, out_text,
                  sim_t2i_v, sim_i2t_v, buf0, buf1,
                  isem0, isem1, osem0, osem1):
    c = lax.axis_index("c")
    s = lax.axis_index("s")
    wid = c * NS + s
    base = wid * RPW

    pltpu.sync_copy(i2t_hbm.at[pl.ds(base, RPW)], sim_i2t_v)
    txt_idx = [_row_argmax(sim_i2t_v, j) for j in range(RPW)]

    moves = []
    for j in range(RPW):
        r = base + j
        moves.append((text_hbm.at[r], out_text.at[r]))
        moves.append((text_hbm.at[txt_idx[j]], out_text.at[B + r]))

    bufs = (buf0, buf1)
    isems = (isem0, isem1)
    osems = (osem0, osem1)
    pending = [None, None]
    for i, (src, dst) in enumerate(moves):
        b = i % 2
        if pending[b] is not None:
            pending[b].wait()
        pltpu.async_copy(src, bufs[b], isems[b]).wait()
        pending[b] = pltpu.async_copy(bufs[b], dst, osems[b])
    for h in pending:
        h.wait()


def _tc_copy_body(img_in, img_out, sem_i):
    i = pl.program_id(0)
    handles = []
    for j in range(4):
        ci = pltpu.make_async_copy(
            img_in.at[:, pl.ds(i * CPB + (CPB // 4) * j, CPB // 4), :],
            img_out.at[:, pl.ds((CPB // 4) * j, CPB // 4), :], sem_i.at[j])
        ci.start()
        handles.append(ci)
    for h in handles:
        h.wait()


def _tc_gather_body(iref, img_in, img_prev, img_out, sem_i):
    i = pl.program_id(0)
    handles = []
    for j in range(RPB):
        n = i * RPB + j
        w = n // RPW
        sl = n % RPW
        hi = pltpu.make_async_copy(
            img_in.at[:, iref[w, sl], :], img_out.at[:, j, :],
            sem_i.at[j])
        hi.start()
        handles.append(hi)
    for h in handles:
        h.wait()


def _out_map(i, iref):
    return (i, 0, 0)


@jax.jit
def kernel(text_embeddings, text_attns, image_embeddings, image_attns,
           sim_t2i, sim_i2t):
    mesh = plsc.VectorSubcoreMesh(
        core_axis_name="c", subcore_axis_name="s",
        num_cores=NC, num_subcores=NS)
    sc_out_type = (
        jax.ShapeDtypeStruct((NW, L), jnp.int32),
        jax.ShapeDtypeStruct((2 * B, LT), jnp.float32),
        jax.ShapeDtypeStruct((2 * B, LI), jnp.float32),
    )
    sc_scratch = [
        pltpu.VMEM((RPW, B), jnp.float32),
        pltpu.VMEM((RPW, B), jnp.float32),
        pltpu.VMEM((L,), jnp.int32),
        pltpu.SemaphoreType.DMA,
    ]
    sc_run = pl.kernel(
        _sc_body, out_type=sc_out_type, mesh=mesh, scratch_types=sc_scratch,
        compiler_params=pltpu.CompilerParams(needs_layout_passes=False),
        name="neg_collector_sc_idx_attn")
    idx_packed, text_attns_all, image_attns_all = sc_run(
        text_attns, image_attns, sim_t2i, sim_i2t)

    sc_txt_run = pl.kernel(
        _sc_text_body,
        out_type=jax.ShapeDtypeStruct((2 * B, LT, D), jnp.float32),
        mesh=mesh,
        scratch_types=[
            pltpu.VMEM((RPW, B), jnp.float32),
            pltpu.VMEM((RPW, B), jnp.float32),
            pltpu.VMEM((LT, D), jnp.float32),
            pltpu.VMEM((LT, D), jnp.float32),
            pltpu.SemaphoreType.DMA,
            pltpu.SemaphoreType.DMA,
            pltpu.SemaphoreType.DMA,
            pltpu.SemaphoreType.DMA,
        ],
        compiler_params=pltpu.CompilerParams(needs_layout_passes=False),
        name="neg_collector_sc_text")
    text_embed_all = sc_txt_run(text_embeddings, sim_t2i, sim_i2t)

    img_t = jnp.transpose(image_embeddings, (1, 0, 2))
    img_half_t = pl.pallas_call(
        _tc_copy_body,
        grid=(B // CPB,),
        in_specs=[pl.BlockSpec(memory_space=pl.ANY)],
        out_specs=[pl.BlockSpec((LI, CPB, D), lambda i: (0, i, 0))],
        out_shape=[jax.ShapeDtypeStruct((LI, 2 * B, D), jnp.float32)],
        scratch_shapes=[pltpu.SemaphoreType.DMA((4,))],
        name="neg_collector_tc_copy",
    )(img_t)[0]

    gather_spec = pltpu.PrefetchScalarGridSpec(
        num_scalar_prefetch=1,
        grid=(B // RPB,),
        in_specs=[
            pl.BlockSpec(memory_space=pl.ANY),
            pl.BlockSpec(memory_space=pl.ANY),
        ],
        out_specs=[
            pl.BlockSpec((LI, RPB, D), lambda i, iref: (0, NCOPY + i, 0)),
        ],
        scratch_shapes=[
            pltpu.SemaphoreType.DMA((RPB,)),
        ],
    )
    image_embed_all_t = pl.pallas_call(
        _tc_gather_body,
        grid_spec=gather_spec,
        out_shape=[jax.ShapeDtypeStruct((LI, 2 * B, D), jnp.float32)],
        input_output_aliases={2: 0},
        name="neg_collector_tc_gather",
    )(idx_packed, img_t, img_half_t)[0]
    image_embed_all = jnp.transpose(image_embed_all_t, (1, 0, 2))

    return (text_embed_all, text_attns_all, image_embed_all, image_attns_all)


# speedup vs baseline: 1.1518x; 1.0327x over previous
import functools

import jax
import jax.numpy as jnp
from jax import lax
from jax.experimental import pallas as pl
from jax.experimental.pallas import tpu as pltpu
from jax.experimental.pallas import tpu_sc as plsc

B = 128
LT = 64
LI = 197
D = 768
NC = 2
NS = 16
NW = NC * NS
RPW = B // NW
L = 16
BIG = 1 << 30
NQ = 8


def _row_argmax(simbuf, j):
    lanes = lax.broadcasted_iota(jnp.int32, (L,), 0)
    best_val = jnp.full((L,), -jnp.inf, dtype=jnp.float32)
    best_idx = jnp.zeros((L,), dtype=jnp.int32)
    for k in range(B // L):
        v = simbuf[j, pl.ds(k * L, L)]
        idxs = lanes + (k * L)
        better = v > best_val
        best_val = jnp.where(better, v, best_val)
        best_idx = jnp.where(better, idxs, best_idx)
    m = jnp.max(best_val)
    cand = jnp.where(best_val == m, best_idx, BIG)
    return jnp.min(cand)


def _sc_body(tattn_hbm, iattn_hbm, t2i_hbm, i2t_hbm,
             idx_out, out_tattn, out_iattn,
             sim_t2i_v, sim_i2t_v, myidx_v, gsem):
    c = lax.axis_index("c")
    s = lax.axis_index("s")
    wid = c * NS + s
    base = wid * RPW

    pltpu.sync_copy(t2i_hbm.at[pl.ds(base, RPW)], sim_t2i_v)
    pltpu.sync_copy(i2t_hbm.at[pl.ds(base, RPW)], sim_i2t_v)

    img_idx = [_row_argmax(sim_t2i_v, j) for j in range(RPW)]
    txt_idx = [_row_argmax(sim_i2t_v, j) for j in range(RPW)]

    lanes = lax.broadcasted_iota(jnp.int32, (L,), 0)
    vec = jnp.zeros((L,), dtype=jnp.int32)
    for j in range(RPW):
        vec = jnp.where(lanes == j, img_idx[j], vec)
        vec = jnp.where(lanes == RPW + j, txt_idx[j], vec)
    myidx_v[...] = vec
    copies = [pltpu.async_copy(myidx_v, idx_out.at[wid], gsem)]

    for j in range(RPW):
        r = base + j
        copies.append(pltpu.async_copy(iattn_hbm.at[r], out_iattn.at[r], gsem))
        copies.append(pltpu.async_copy(tattn_hbm.at[r], out_tattn.at[r], gsem))
        copies.append(
            pltpu.async_copy(iattn_hbm.at[img_idx[j]], out_iattn.at[B + r], gsem))
        copies.append(
            pltpu.async_copy(tattn_hbm.at[txt_idx[j]], out_tattn.at[B + r], gsem))
    for cp in copies:
        cp.wait()


RPB = 32
CPB = 32
NCOPY = B // RPB


def _sc_text_body(text_hbm, t2i_hbm, i2t_hbm, out_text,
                  sim_t2i_v, sim_i2t_v, buf0, buf1,
                  isem0, isem1, osem0, osem1):
    c = lax.axis_index("c")
    s = lax.axis_index("s")
    wid = c * NS + s
    base = wid * RPW

    pltpu.sync_copy(i2t_hbm.at[pl.ds(base, RPW)], sim_i2t_v)
    txt_idx = [_row_argmax(sim_i2t_v, j) for j in range(RPW)]

    moves = []
    for j in range(RPW):
        r = base + j
        moves.append((text_hbm.at[r], out_text.at[r]))
        moves.append((text_hbm.at[txt_idx[j]], out_text.at[B + r]))

    bufs = (buf0, buf1)
    isems = (isem0, isem1)
    osems = (osem0, osem1)
    pending = [None, None]
    for i, (src, dst) in enumerate(moves):
        b = i % 2
        if pending[b] is not None:
            pending[b].wait()
        pltpu.async_copy(src, bufs[b], isems[b]).wait()
        pending[b] = pltpu.async_copy(bufs[b], dst, osems[b])
    for h in pending:
        h.wait()


def _tc_copy_body(img_in, img_out, sem_i):
    i = pl.program_id(0)
    handles = []
    for j in range(4):
        ci = pltpu.make_async_copy(
            img_in.at[:, pl.ds(i * CPB + (CPB // 4) * j, CPB // 4), :],
            img_out.at[:, pl.ds((CPB // 4) * j, CPB // 4), :], sem_i.at[j])
        ci.start()
        handles.append(ci)
    for h in handles:
        h.wait()


def _tc_gather_body(iref, img_in, img_prev, img_out, sem_i):
    i = pl.program_id(0)
    handles = []
    for j in range(RPB):
        n = i * RPB + j
        w = n // RPW
        sl = n % RPW
        hi = pltpu.make_async_copy(
            img_in.at[:, iref[w, sl], :], img_out.at[:, j, :],
            sem_i.at[j])
        hi.start()
        handles.append(hi)
    for h in handles:
        h.wait()


def _out_map(i, iref):
    return (i, 0, 0)


@jax.jit
def kernel(text_embeddings, text_attns, image_embeddings, image_attns,
           sim_t2i, sim_i2t):
    mesh = plsc.VectorSubcoreMesh(
        core_axis_name="c", subcore_axis_name="s",
        num_cores=NC, num_subcores=NS)
    sc_out_type = (
        jax.ShapeDtypeStruct((NW, L), jnp.int32),
        jax.ShapeDtypeStruct((2 * B, LT), jnp.float32),
        jax.ShapeDtypeStruct((2 * B, LI), jnp.float32),
    )
    sc_scratch = [
        pltpu.VMEM((RPW, B), jnp.float32),
        pltpu.VMEM((RPW, B), jnp.float32),
        pltpu.VMEM((L,), jnp.int32),
        pltpu.SemaphoreType.DMA,
    ]
    sc_run = pl.kernel(
        _sc_body, out_type=sc_out_type, mesh=mesh, scratch_types=sc_scratch,
        compiler_params=pltpu.CompilerParams(needs_layout_passes=False),
        name="neg_collector_sc_idx_attn")
    idx_packed, text_attns_all, image_attns_all = sc_run(
        text_attns, image_attns, sim_t2i, sim_i2t)

    sc_txt_run = pl.kernel(
        _sc_text_body,
        out_type=jax.ShapeDtypeStruct((2 * B, LT, D), jnp.float32),
        mesh=mesh,
        scratch_types=[
            pltpu.VMEM((RPW, B), jnp.float32),
            pltpu.VMEM((RPW, B), jnp.float32),
            pltpu.VMEM((LT, D), jnp.float32),
            pltpu.VMEM((LT, D), jnp.float32),
            pltpu.SemaphoreType.DMA,
            pltpu.SemaphoreType.DMA,
            pltpu.SemaphoreType.DMA,
            pltpu.SemaphoreType.DMA,
        ],
        compiler_params=pltpu.CompilerParams(needs_layout_passes=False),
        name="neg_collector_sc_text")
    text_embed_all = sc_txt_run(text_embeddings, sim_t2i, sim_i2t)

    img_t = jnp.transpose(image_embeddings, (1, 0, 2))
    img_half_t = pl.pallas_call(
        _tc_copy_body,
        grid=(B // CPB,),
        in_specs=[pl.BlockSpec(memory_space=pl.ANY)],
        out_specs=[pl.BlockSpec((LI, CPB, D), lambda i: (0, i, 0))],
        out_shape=[jax.ShapeDtypeStruct((LI, 2 * B, D), jnp.float32)],
        scratch_shapes=[pltpu.SemaphoreType.DMA((4,))],
        name="neg_collector_tc_copy",
    )(img_t)[0]

    gather_spec = pltpu.PrefetchScalarGridSpec(
        num_scalar_prefetch=1,
        grid=(B // RPB,),
        in_specs=[
            pl.BlockSpec(memory_space=pl.ANY),
            pl.BlockSpec(memory_space=pl.ANY),
        ],
        out_specs=[
            pl.BlockSpec((LI, RPB, D), lambda i, iref: (0, NCOPY + i, 0)),
        ],
        scratch_shapes=[
            pltpu.SemaphoreType.DMA((RPB,)),
        ],
    )
    image_embed_all_t = pl.pallas_call(
        _tc_gather_body,
        grid_spec=gather_spec,
        out_shape=[jax.ShapeDtypeStruct((LI, 2 * B, D), jnp.float32)],
        input_output_aliases={2: 0},
        name="neg_collector_tc_gather",
    )(idx_packed, img_t, img_half_t)[0]
    image_embed_all = jnp.transpose(image_embed_all_t, (1, 0, 2))

    return (text_embed_all, text_attns_all, image_embed_all, image_attns_all)
